# baseline (speedup 1.0000x reference)
import functools

import jax
import jax.numpy as jnp
from jax import lax
from jax.experimental import pallas as pl
from jax.experimental.pallas import tpu as pltpu
import jax.experimental.pallas.tpu_sc as plsc

B, N, D, K, C = 8, 65536, 128, 64, 2
BN = 4096
NB = N // BN
NEG = -1e9


def _scorer_body(inst_ref, mask_ref, w1_ref, aux_ref, b2_ref, out_ref):
  x16 = inst_ref[0].astype(jnp.bfloat16)
  w116 = w1_ref[...].astype(jnp.bfloat16)
  h = jnp.maximum(jnp.dot(x16, w116, preferred_element_type=jnp.float32)
                  + aux_ref[0][None, :], 0.0)
  h16 = h.astype(jnp.bfloat16)
  w216 = aux_ref[1].astype(jnp.bfloat16)
  s = jnp.dot(h16, w216[:, None],
              preferred_element_type=jnp.float32)[:, 0] + b2_ref[0]
  s = jnp.where(mask_ref[0, 0, 0] == 0, NEG, s)
  out_ref[0, 0, 0] = s


def _scores_tc(instances, mask4, W1, aux, b2):
  return pl.pallas_call(
      _scorer_body,
      grid=(B, NB),
      in_specs=[
          pl.BlockSpec((1, BN, D), lambda b, n: (b, n, 0)),
          pl.BlockSpec((1, 1, 1, BN), lambda b, n: (b, n, 0, 0)),
          pl.BlockSpec((D, D // 2), lambda b, n: (0, 0)),
          pl.BlockSpec((2, D // 2), lambda b, n: (0, 0)),
          pl.BlockSpec(memory_space=pltpu.SMEM),
      ],
      out_specs=pl.BlockSpec((1, 1, 1, BN), lambda b, n: (b, n, 0, 0)),
      out_shape=jax.ShapeDtypeStruct((B, NB, 1, BN), jnp.float32),
  )(instances, mask4, W1, aux, b2)


def _finish_body(s_ref, bf_ref, wct_ref, bc_ref, att_ref, log_ref):
  s = s_ref[0]
  m = jnp.max(s)
  e = jnp.exp(s - m)
  att_ref[0] = e * (1.0 / jnp.sum(e))
  bf = bf_ref[0, 0].astype(jnp.bfloat16).astype(jnp.float32)
  wct = wct_ref[...].astype(jnp.bfloat16).astype(jnp.float32)
  logits = jnp.sum(wct * bf[None, :], axis=1) + bc_ref[0]
  log_ref[0, 0] = logits


def _finish_tc(scores3, bag_features, WcT, bc2):
  return pl.pallas_call(
      _finish_body,
      grid=(B,),
      in_specs=[
          pl.BlockSpec((1, N // D, D), lambda b: (b, 0, 0)),
          pl.BlockSpec((1, 1, D), lambda b: (b, 0, 0)),
          pl.BlockSpec((C, D), lambda b: (0, 0)),
          pl.BlockSpec((1, C), lambda b: (0, 0)),
      ],
      out_specs=[
          pl.BlockSpec((1, N // D, D), lambda b: (b, 0, 0)),
          pl.BlockSpec((1, 1, C), lambda b: (b, 0, 0)),
      ],
      out_shape=[
          jax.ShapeDtypeStruct((B, N // D, D), jnp.float32),
          jax.ShapeDtypeStruct((B, 1, C), jnp.float32),
      ],
  )(scores3, bag_features.reshape(B, 1, D), WcT, bc2)


GROUPS = 64
GSZ = N // GROUPS
CHUNK = GSZ // 16
SENT = -3.0e38


def _topk_body(scores_hbm, inst_hbm, topk_hbm, bf_hbm,
               sv, cm, gsm, idxout, gidx, rows, bfout, sem):
  cid = lax.axis_index("c")
  sid = lax.axis_index("s")
  bag = cid * 4 + sid // 4
  iota = lax.broadcasted_iota(jnp.int32, (16,), 0)
  iota16 = iota * 16
  lane0 = iota == 0
  BIGI = jnp.int32(1 << 30)

  def full_i(x):
    return jnp.full((16,), x, jnp.int32)

  def full_f(x):
    return jnp.full((16,), x, jnp.float32)

  @pl.when(sid % 4 == 0)
  def _leader():
    pltpu.sync_copy(scores_hbm.at[bag], sv)

    gsmv = [full_f(SENT) for _ in range(4)]
    for g in range(GROUPS):
      def _init(t, acc):
        base = g * GSZ + t * 64
        for u in range(4):
          acc = jnp.maximum(acc, plsc.load_gather(sv, [base + u * 16 + iota]))
        return acc
      acc = lax.fori_loop(0, CHUNK // 4, _init, full_f(SENT))
      cm[pl.ds(g * 16, 16)] = acc
      gsmv[g // 16] = jnp.where(iota == (g % 16), jnp.max(acc), gsmv[g // 16])
    for i in range(4):
      gsm[pl.ds(i * 16, 16)] = gsmv[i]

    def _extract(t, carry):
      g0 = gsm[pl.ds(0, 16)]
      g1 = gsm[pl.ds(16, 16)]
      g2 = gsm[pl.ds(32, 16)]
      g3 = gsm[pl.ds(48, 16)]
      gm = jnp.max(jnp.maximum(jnp.maximum(g0, g1), jnp.maximum(g2, g3)))
      pm = jnp.minimum(
          jnp.minimum(jnp.where(g0 == gm, iota, BIGI),
                      jnp.where(g1 == gm, iota + 16, BIGI)),
          jnp.minimum(jnp.where(g2 == gm, iota + 32, BIGI),
                      jnp.where(g3 == gm, iota + 48, BIGI)))
      gstar = jnp.min(pm)
      cmrow = plsc.load_gather(cm, [gstar * 16 + iota])
      lstar = jnp.min(jnp.where(cmrow == gm, iota, BIGI))
      base = gstar * GSZ + lstar
      pacc = full_i(BIGI)
      for u in range(4):
        idxv = base + u * 256 + iota16
        vals = plsc.load_gather(sv, [idxv])
        pacc = jnp.minimum(pacc, jnp.where(vals == gm, idxv, BIGI))
      p = jnp.min(pacc)
      plsc.store_scatter(sv, [full_i(p)], full_f(SENT), mask=lane0)
      macc = full_f(SENT)
      for u in range(4):
        macc = jnp.maximum(macc,
                           plsc.load_gather(sv, [base + u * 256 + iota16]))
      newmax = jnp.max(macc)
      plsc.store_scatter(cm, [full_i(gstar * 16 + lstar)], full_f(newmax),
                         mask=lane0)
      newrow = jnp.where(iota == lstar, newmax, cmrow)
      plsc.store_scatter(gsm, [full_i(gstar)], full_f(jnp.max(newrow)),
                         mask=lane0)
      plsc.store_scatter(idxout, [full_i(t)], full_i(p), mask=lane0)
      plsc.store_scatter(gidx, [full_i(t)], full_i(p + bag * N), mask=lane0)
      return carry

    lax.fori_loop(0, K, _extract, jnp.int32(0))

    pltpu.async_copy(inst_hbm.at[gidx], rows, sem).wait()

    def _pool(r, accs):
      rv = full_i(r)
      return tuple(
          accs[c] + plsc.load_gather(rows, [rv, c * 16 + iota])
          for c in range(8))
    accs = lax.fori_loop(0, K, _pool, tuple(full_f(0.0) for _ in range(8)))
    for c in range(8):
      bfout[pl.ds(c * 16, 16)] = accs[c] * (1.0 / K)

    pltpu.sync_copy(idxout, topk_hbm.at[bag])
    pltpu.sync_copy(bfout, bf_hbm.at[bag])


def _topk_sc(scores, inst_flat):
  mesh = plsc.VectorSubcoreMesh(core_axis_name="c", subcore_axis_name="s",
                                num_cores=2, num_subcores=16)
  f = pl.kernel(
      _topk_body,
      out_type=[
          jax.ShapeDtypeStruct((B, K), jnp.int32),
          jax.ShapeDtypeStruct((B, D), jnp.float32),
      ],
      mesh=mesh,
      scratch_types=[
          pltpu.VMEM((N,), jnp.float32),
          pltpu.VMEM((GROUPS * 16,), jnp.float32),
          pltpu.VMEM((GROUPS,), jnp.float32),
          pltpu.VMEM((K,), jnp.int32),
          pltpu.VMEM((K,), jnp.int32),
          pltpu.VMEM((K, D), jnp.float32),
          pltpu.VMEM((D,), jnp.float32),
          pltpu.SemaphoreType.DMA,
      ],
      compiler_params=pltpu.CompilerParams(use_tc_tiling_on_sc=False,
                                           needs_layout_passes=False),
  )
  return f(scores, inst_flat)


def kernel(instances, mask, W1, b1, W2, b2, Wc, bc):
  instances = jnp.asarray(instances)
  if instances.ndim == 2:
    instances = instances[None]
  mask4 = mask.reshape(B, NB, 1, BN)
  aux = jnp.stack([b1, W2[:, 0]])
  scores4 = _scores_tc(instances, mask4, W1, aux, b2)
  scores = scores4.reshape(B, N)

  topk_indices, bag_features = _topk_sc(scores, instances.reshape(B * N, D))

  att3, logits = _finish_tc(scores.reshape(B, N // D, D), bag_features,
                            Wc.T, bc.reshape(1, C))
  attention_weights = att3.reshape(B, N)
  return (logits.reshape(B, C), attention_weights, topk_indices, bag_features)


# speedup vs baseline: 2.0658x; 2.0658x over previous
import functools

import jax
import jax.numpy as jnp
from jax import lax
from jax.experimental import pallas as pl
from jax.experimental.pallas import tpu as pltpu
import jax.experimental.pallas.tpu_sc as plsc

B, N, D, K, C = 8, 65536, 128, 64, 2
BN = 8192
NB = N // BN
NEG = -1e9

HB = B // 2
SPB = 32 // HB
SHARD = N // SPB
GROUPS = 64
GSZ = N // GROUPS
CHUNK = GSZ // 16
GPS = GROUPS // SPB
PART = GPS * 16 + 16
SENT = -3.0e38


def _scorer_body(inst_ref, mask_ref, w1_ref, aux_ref, b2_ref, out_ref):
  x = inst_ref[0]
  h = jnp.maximum(jnp.dot(x, w1_ref[...], preferred_element_type=jnp.float32)
                  + aux_ref[0][None, :], 0.0)
  s2 = lax.dot_general(aux_ref[1][None, :], h, (((1,), (1,)), ((), ())),
                       preferred_element_type=jnp.float32)
  s = s2[0] + b2_ref[0]
  s = jnp.where(mask_ref[0, 0, 0] == 0, NEG, s)
  out_ref[0, 0, 0] = s


def _scores_tc(instances, mask4, W1, aux, b2, b0):
  return pl.pallas_call(
      _scorer_body,
      grid=(HB, NB),
      in_specs=[
          pl.BlockSpec((1, BN, D), lambda b, n: (b0 + b, n, 0)),
          pl.BlockSpec((1, 1, 1, BN), lambda b, n: (b0 + b, n, 0, 0)),
          pl.BlockSpec((D, D // 2), lambda b, n: (0, 0)),
          pl.BlockSpec((2, D // 2), lambda b, n: (0, 0)),
          pl.BlockSpec(memory_space=pltpu.SMEM),
      ],
      out_specs=pl.BlockSpec((1, 1, 1, BN), lambda b, n: (b, n, 0, 0)),
      out_shape=jax.ShapeDtypeStruct((HB, NB, 1, BN), jnp.float32),
  )(instances, mask4, W1, aux, b2)


def _finish_body(s_ref, att_ref):
  s = s_ref[0]
  m = jnp.max(s)
  e = jnp.exp(s - m)
  att_ref[0] = e * (1.0 / jnp.sum(e))


def _finish_tc(scores3):
  return pl.pallas_call(
      _finish_body,
      grid=(HB,),
      in_specs=[
          pl.BlockSpec((1, N // D, D), lambda b: (b, 0, 0)),
      ],
      out_specs=pl.BlockSpec((1, N // D, D), lambda b: (b, 0, 0)),
      out_shape=jax.ShapeDtypeStruct((HB, N // D, D), jnp.float32),
  )(scores3)


def _round_bf16(x):
  b = plsc.bitcast(x, jnp.int32)
  r = (b + 0x7FFF + ((b >> 16) & 1)) & jnp.int32(-65536)
  return plsc.bitcast(r, jnp.float32)


def _make_topk_body(b0):
  def _topk_body(scores_hbm, inst_hbm, wcb_hbm, topk_hbm, bf_hbm, log_hbm,
                 sv, qsv, cm, gsm, part, shared, idxout, gidx, rows, bfout,
                 wcb, logout, sem, sem2):
    cid = lax.axis_index("c")
    sid = lax.axis_index("s")
    q = sid % SPB
    bagl = sid // SPB
    bag = cid * (HB // 2) + bagl
    iota = lax.broadcasted_iota(jnp.int32, (16,), 0)
    iota16 = iota * 16
    lane0 = iota == 0
    BIGI = jnp.int32(1 << 30)

    def full_i(x):
      return jnp.full((16,), x, jnp.int32)

    def full_f(x):
      return jnp.full((16,), x, jnp.float32)

    @pl.when(q == 0)
    def _():
      pltpu.async_copy(scores_hbm.at[bag], sv, sem2)

    pltpu.sync_copy(scores_hbm.at[bag, pl.ds(q * SHARD, SHARD)], qsv)
    pg = full_f(SENT)
    for gg in range(GPS):
      def _init(t, acc):
        base = gg * GSZ + t * 64
        for u in range(4):
          acc = jnp.maximum(acc,
                            plsc.load_gather(qsv, [base + u * 16 + iota]))
        return acc
      acc = lax.fori_loop(0, CHUNK // 4, _init, full_f(SENT))
      part[pl.ds(gg * 16, 16)] = acc
      pg = jnp.where(iota == gg, jnp.max(acc), pg)
    part[pl.ds(GPS * 16, 16)] = pg

    pltpu.sync_copy(part, shared.at[sid])
    plsc.subcore_barrier()

    @pl.when(q == 0)
    def _leader():
      for qq in range(SPB):
        src = bagl * SPB + qq
        pltpu.sync_copy(shared.at[src, pl.ds(0, GPS * 16)],
                        cm.at[pl.ds(qq * GPS * 16, GPS * 16)])
        pltpu.sync_copy(shared.at[src, pl.ds(GPS * 16, GPS)],
                        gsm.at[pl.ds(qq * GPS, GPS)])
      pltpu.make_async_copy(scores_hbm.at[bag], sv, sem2).wait()

      def _extract(t, carry):
        g0 = gsm[pl.ds(0, 16)]
        g1 = gsm[pl.ds(16, 16)]
        g2 = gsm[pl.ds(32, 16)]
        g3 = gsm[pl.ds(48, 16)]
        gm = jnp.max(jnp.maximum(jnp.maximum(g0, g1), jnp.maximum(g2, g3)))
        pm = jnp.minimum(
            jnp.minimum(jnp.where(g0 == gm, iota, BIGI),
                        jnp.where(g1 == gm, iota + 16, BIGI)),
            jnp.minimum(jnp.where(g2 == gm, iota + 32, BIGI),
                        jnp.where(g3 == gm, iota + 48, BIGI)))
        gstar = jnp.min(pm)
        cmrow = plsc.load_gather(cm, [gstar * 16 + iota])
        lstar = jnp.min(jnp.where(cmrow == gm, iota, BIGI))
        base = gstar * GSZ + lstar
        pacc = full_i(BIGI)
        vals_l, idx_l = [], []
        for u in range(4):
          idxv = base + u * 256 + iota16
          vals = plsc.load_gather(sv, [idxv])
          vals_l.append(vals)
          idx_l.append(idxv)
          pacc = jnp.minimum(pacc, jnp.where(vals == gm, idxv, BIGI))
        p = jnp.min(pacc)
        plsc.store_scatter(sv, [full_i(p)], full_f(SENT), mask=lane0)
        macc = full_f(SENT)
        for u in range(4):
          macc = jnp.maximum(
              macc, jnp.where(idx_l[u] == p, full_f(SENT), vals_l[u]))
        newmax = jnp.max(macc)
        plsc.store_scatter(cm, [full_i(gstar * 16 + lstar)], full_f(newmax),
                           mask=lane0)
        newrow = jnp.where(iota == lstar, newmax, cmrow)
        plsc.store_scatter(gsm, [full_i(gstar)], full_f(jnp.max(newrow)),
                           mask=lane0)
        plsc.store_scatter(idxout, [full_i(t)], full_i(p), mask=lane0)
        plsc.store_scatter(gidx, [full_i(t)],
                           full_i(p + (b0 + bag) * N), mask=lane0)
        return carry

      lax.fori_loop(0, K, _extract, jnp.int32(0))

      pltpu.async_copy(inst_hbm.at[gidx], rows, sem).wait()

      def _pool(r, accs):
        rv = full_i(r)
        return tuple(
            accs[c] + plsc.load_gather(rows, [rv, c * 16 + iota])
            for c in range(8))
      accs = lax.fori_loop(0, K, _pool, tuple(full_f(0.0) for _ in range(8)))
      bfv = [accs[c] * (1.0 / K) for c in range(8)]
      for c in range(8):
        bfout[pl.ds(c * 16, 16)] = bfv[c]

      pltpu.sync_copy(wcb_hbm, wcb)
      lacc0 = full_f(0.0)
      lacc1 = full_f(0.0)
      for c in range(8):
        bfr = _round_bf16(bfv[c])
        lacc0 = lacc0 + bfr * _round_bf16(wcb[pl.ds(c * 16, 16)])
        lacc1 = lacc1 + bfr * _round_bf16(wcb[pl.ds(128 + c * 16, 16)])
      l0 = jnp.sum(lacc0) + wcb[pl.ds(256, 16)][0]
      l1 = jnp.sum(lacc1) + wcb[pl.ds(256, 16)][1]
      lvec = jnp.where(iota == 0, full_f(l0),
                       jnp.where(iota == 1, full_f(l1), full_f(0.0)))
      logout[pl.ds(0, 16)] = lvec

      pltpu.sync_copy(idxout, topk_hbm.at[bag])
      pltpu.sync_copy(bfout, bf_hbm.at[bag])
      pltpu.sync_copy(logout, log_hbm.at[bag])

  return _topk_body


def _topk_sc(scores, inst_flat, wcb, b0):
  mesh = plsc.VectorSubcoreMesh(core_axis_name="c", subcore_axis_name="s",
                                num_cores=2, num_subcores=16)
  f = pl.kernel(
      _make_topk_body(b0),
      out_type=[
          jax.ShapeDtypeStruct((HB, K), jnp.int32),
          jax.ShapeDtypeStruct((HB, D), jnp.float32),
          jax.ShapeDtypeStruct((HB, 16), jnp.float32),
      ],
      mesh=mesh,
      scratch_types=[
          pltpu.VMEM((N,), jnp.float32),
          pltpu.VMEM((SHARD,), jnp.float32),
          pltpu.VMEM((GROUPS * 16,), jnp.float32),
          pltpu.VMEM((GROUPS,), jnp.float32),
          pltpu.VMEM((PART,), jnp.float32),
          pltpu.VMEM_SHARED((16, PART), jnp.float32),
          pltpu.VMEM((K,), jnp.int32),
          pltpu.VMEM((K,), jnp.int32),
          pltpu.VMEM((K, D), jnp.float32),
          pltpu.VMEM((D,), jnp.float32),
          pltpu.VMEM((2 * D + 16,), jnp.float32),
          pltpu.VMEM((16,), jnp.float32),
          pltpu.SemaphoreType.DMA,
          pltpu.SemaphoreType.DMA,
      ],
      compiler_params=pltpu.CompilerParams(use_tc_tiling_on_sc=False,
                                           needs_layout_passes=False),
  )
  return f(scores, inst_flat, wcb)


def kernel(instances, mask, W1, b1, W2, b2, Wc, bc):
  instances = jnp.asarray(instances)
  if instances.ndim == 2:
    instances = instances[None]
  mask4 = mask.reshape(B, NB, 1, BN)
  aux = jnp.stack([b1, W2[:, 0]])
  wcb = jnp.concatenate([Wc.T.reshape(-1), bc,
                         jnp.zeros((16 - C,), jnp.float32)])
  inst_flat = instances.reshape(B * N, D)

  tks, bfs, lgs, atts = [], [], [], []
  for b0 in (0, HB):
    scores4 = _scores_tc(instances, mask4, W1, aux, b2, b0)
    scores = scores4.reshape(HB, N)
    tk, bf, lg = _topk_sc(scores, inst_flat, wcb, b0)
    att3 = _finish_tc(scores.reshape(HB, N // D, D))
    tks.append(tk)
    bfs.append(bf)
    lgs.append(lg)
    atts.append(att3.reshape(HB, N))

  topk_indices = jnp.concatenate(tks)
  bag_features = jnp.concatenate(bfs)
  logits = jnp.concatenate(lgs)[:, :C]
  attention_weights = jnp.concatenate(atts)
  return (logits, attention_weights, topk_indices, bag_features)


# speedup vs baseline: 2.1247x; 1.0285x over previous
import functools

import jax
import jax.numpy as jnp
from jax import lax
from jax.experimental import pallas as pl
from jax.experimental.pallas import tpu as pltpu
import jax.experimental.pallas.tpu_sc as plsc

B, N, D, K, C = 8, 65536, 128, 64, 2
BN = 8192
NB = N // BN
NEG = -1e9


def _scorer_body(inst_ref, mask_ref, w1_ref, aux_ref, b2_ref, out_ref):
  x = inst_ref[0]
  h = jnp.maximum(jnp.dot(x, w1_ref[...], preferred_element_type=jnp.float32)
                  + aux_ref[0][None, :], 0.0)
  s2 = lax.dot_general(aux_ref[1][None, :], h, (((1,), (1,)), ((), ())),
                       preferred_element_type=jnp.float32)
  s = s2[0] + b2_ref[0]
  s = jnp.where(mask_ref[0, 0, 0] == 0, NEG, s)
  out_ref[0, 0, 0] = s


def _scores_tc(instances, mask4, W1, aux, b2):
  return pl.pallas_call(
      _scorer_body,
      grid=(B, NB),
      in_specs=[
          pl.BlockSpec((1, BN, D), lambda b, n: (b, n, 0)),
          pl.BlockSpec((1, 1, 1, BN), lambda b, n: (b, n, 0, 0)),
          pl.BlockSpec((D, D // 2), lambda b, n: (0, 0)),
          pl.BlockSpec((2, D // 2), lambda b, n: (0, 0)),
          pl.BlockSpec(memory_space=pltpu.SMEM),
      ],
      out_specs=pl.BlockSpec((1, 1, 1, BN), lambda b, n: (b, n, 0, 0)),
      out_shape=jax.ShapeDtypeStruct((B, NB, 1, BN), jnp.float32),
  )(instances, mask4, W1, aux, b2)


def _finish_body(s_ref, att_ref):
  s = s_ref[0]
  m = jnp.max(s)
  e = jnp.exp(s - m)
  att_ref[0] = e * (1.0 / jnp.sum(e))


def _finish_tc(scores3):
  return pl.pallas_call(
      _finish_body,
      grid=(B,),
      in_specs=[
          pl.BlockSpec((1, N // D, D), lambda b: (b, 0, 0)),
      ],
      out_specs=pl.BlockSpec((1, N // D, D), lambda b: (b, 0, 0)),
      out_shape=jax.ShapeDtypeStruct((B, N // D, D), jnp.float32),
  )(scores3)


GROUPS = 64
GSZ = N // GROUPS
CHUNK = GSZ // 16
SENT = -3.0e38


def _round_bf16(x):
  b = plsc.bitcast(x, jnp.int32)
  r = (b + 0x7FFF + ((b >> 16) & 1)) & jnp.int32(-65536)
  return plsc.bitcast(r, jnp.float32)


def _topk_body(scores_hbm, inst_hbm, wcb_hbm, topk_hbm, bf_hbm, log_hbm,
               sv, qsv, cm, gsm, part, shared, merged, idxout, gidx, rows,
               bfout, wcb, logout, sem, sem2):
  cid = lax.axis_index("c")
  sid = lax.axis_index("s")
  q = sid % 4
  bagl = sid // 4
  bag = cid * 4 + bagl
  iota = lax.broadcasted_iota(jnp.int32, (16,), 0)
  iota16 = iota * 16
  lane0 = iota == 0
  BIGI = jnp.int32(1 << 30)

  def full_i(x):
    return jnp.full((16,), x, jnp.int32)

  def full_f(x):
    return jnp.full((16,), x, jnp.float32)

  def _quarter_cm(src):
    pg = full_f(SENT)
    for gg in range(16):
      def _init(t, acc):
        base = gg * GSZ + t * 64
        for u in range(4):
          acc = jnp.maximum(acc, plsc.load_gather(src, [base + u * 16 + iota]))
        return acc
      acc = lax.fori_loop(0, CHUNK // 4, _init, full_f(SENT))
      part[pl.ds(gg * 16, 16)] = acc
      pg = jnp.where(iota == gg, jnp.max(acc), pg)
    part[pl.ds(256, 16)] = pg

  @pl.when(q == 0)
  def _():
    pltpu.async_copy(scores_hbm.at[bag], sv, sem2)

  pltpu.sync_copy(scores_hbm.at[bag, pl.ds(q * (N // 4), N // 4)], qsv)
  _quarter_cm(qsv)

  pltpu.sync_copy(part, shared.at[sid])
  plsc.subcore_barrier()

  @pl.when(q == 0)
  def _leader():
    pltpu.sync_copy(shared.at[pl.ds(bagl * 4, 4)], merged)
    for qq in range(4):
      for i in range(16):
        cm[pl.ds(qq * 256 + i * 16, 16)] = merged[qq, pl.ds(i * 16, 16)]
      gsm[pl.ds(qq * 16, 16)] = merged[qq, pl.ds(256, 16)]
    pltpu.make_async_copy(scores_hbm.at[bag], sv, sem2).wait()

    def _extract(t, carry):
      g0 = gsm[pl.ds(0, 16)]
      g1 = gsm[pl.ds(16, 16)]
      g2 = gsm[pl.ds(32, 16)]
      g3 = gsm[pl.ds(48, 16)]
      gm = jnp.max(jnp.maximum(jnp.maximum(g0, g1), jnp.maximum(g2, g3)))
      pm = jnp.minimum(
          jnp.minimum(jnp.where(g0 == gm, iota, BIGI),
                      jnp.where(g1 == gm, iota + 16, BIGI)),
          jnp.minimum(jnp.where(g2 == gm, iota + 32, BIGI),
                      jnp.where(g3 == gm, iota + 48, BIGI)))
      gstar = jnp.min(pm)
      cmrow = plsc.load_gather(cm, [gstar * 16 + iota])
      lstar = jnp.min(jnp.where(cmrow == gm, iota, BIGI))
      base = gstar * GSZ + lstar
      pacc = full_i(BIGI)
      vals_l, idx_l = [], []
      for u in range(4):
        idxv = base + u * 256 + iota16
        vals = plsc.load_gather(sv, [idxv])
        vals_l.append(vals)
        idx_l.append(idxv)
        pacc = jnp.minimum(pacc, jnp.where(vals == gm, idxv, BIGI))
      p = jnp.min(pacc)
      plsc.store_scatter(sv, [full_i(p)], full_f(SENT), mask=lane0)
      macc = full_f(SENT)
      for u in range(4):
        macc = jnp.maximum(macc,
                           jnp.where(idx_l[u] == p, full_f(SENT), vals_l[u]))
      newmax = jnp.max(macc)
      plsc.store_scatter(cm, [full_i(gstar * 16 + lstar)], full_f(newmax),
                         mask=lane0)
      newrow = jnp.where(iota == lstar, newmax, cmrow)
      plsc.store_scatter(gsm, [full_i(gstar)], full_f(jnp.max(newrow)),
                         mask=lane0)
      plsc.store_scatter(idxout, [full_i(t)], full_i(p), mask=lane0)
      plsc.store_scatter(gidx, [full_i(t)], full_i(p + bag * N), mask=lane0)
      return carry

    lax.fori_loop(0, K, _extract, jnp.int32(0))

    pltpu.async_copy(inst_hbm.at[gidx], rows, sem).wait()

    def _pool(r, accs):
      rv = full_i(r)
      return tuple(
          accs[c] + plsc.load_gather(rows, [rv, c * 16 + iota])
          for c in range(8))
    accs = lax.fori_loop(0, K, _pool, tuple(full_f(0.0) for _ in range(8)))
    bfv = [accs[c] * (1.0 / K) for c in range(8)]
    for c in range(8):
      bfout[pl.ds(c * 16, 16)] = bfv[c]

    pltpu.sync_copy(wcb_hbm, wcb)
    lacc0 = full_f(0.0)
    lacc1 = full_f(0.0)
    for c in range(8):
      bfr = _round_bf16(bfv[c])
      lacc0 = lacc0 + bfr * _round_bf16(wcb[pl.ds(c * 16, 16)])
      lacc1 = lacc1 + bfr * _round_bf16(wcb[pl.ds(128 + c * 16, 16)])
    l0 = jnp.sum(lacc0) + wcb[pl.ds(256, 16)][0]
    l1 = jnp.sum(lacc1) + wcb[pl.ds(256, 16)][1]
    lvec = jnp.where(iota == 0, full_f(l0),
                     jnp.where(iota == 1, full_f(l1), full_f(0.0)))
    logout[pl.ds(0, 16)] = lvec

    pltpu.sync_copy(idxout, topk_hbm.at[bag])
    pltpu.sync_copy(bfout, bf_hbm.at[bag])
    pltpu.sync_copy(logout, log_hbm.at[bag])


def _topk_sc(scores, inst_flat, wcb):
  mesh = plsc.VectorSubcoreMesh(core_axis_name="c", subcore_axis_name="s",
                                num_cores=2, num_subcores=16)
  f = pl.kernel(
      _topk_body,
      out_type=[
          jax.ShapeDtypeStruct((B, K), jnp.int32),
          jax.ShapeDtypeStruct((B, D), jnp.float32),
          jax.ShapeDtypeStruct((B, 16), jnp.float32),
      ],
      mesh=mesh,
      scratch_types=[
          pltpu.VMEM((N,), jnp.float32),
          pltpu.VMEM((N // 4,), jnp.float32),
          pltpu.VMEM((GROUPS * 16,), jnp.float32),
          pltpu.VMEM((GROUPS,), jnp.float32),
          pltpu.VMEM((272,), jnp.float32),
          pltpu.VMEM_SHARED((16, 272), jnp.float32),
          pltpu.VMEM((4, 272), jnp.float32),
          pltpu.VMEM((K,), jnp.int32),
          pltpu.VMEM((K,), jnp.int32),
          pltpu.VMEM((K, D), jnp.float32),
          pltpu.VMEM((D,), jnp.float32),
          pltpu.VMEM((2 * D + 16,), jnp.float32),
          pltpu.VMEM((16,), jnp.float32),
          pltpu.SemaphoreType.DMA,
          pltpu.SemaphoreType.DMA,
      ],
      compiler_params=pltpu.CompilerParams(use_tc_tiling_on_sc=False,
                                           needs_layout_passes=False),
  )
  return f(scores, inst_flat, wcb)


def kernel(instances, mask, W1, b1, W2, b2, Wc, bc):
  instances = jnp.asarray(instances)
  if instances.ndim == 2:
    instances = instances[None]
  mask4 = mask.reshape(B, NB, 1, BN)
  aux = jnp.stack([b1, W2[:, 0]])
  scores4 = _scores_tc(instances, mask4, W1, aux, b2)
  scores = scores4.reshape(B, N)

  wcb = jnp.concatenate([Wc.T.reshape(-1), bc,
                         jnp.zeros((16 - C,), jnp.float32)])
  topk_indices, bag_features, logpad = _topk_sc(
      scores, instances.reshape(B * N, D), wcb)

  att3 = _finish_tc(scores.reshape(B, N // D, D))
  attention_weights = att3.reshape(B, N)
  return (logpad[:, :C], attention_weights, topk_indices, bag_features)


# speedup vs baseline: 2.4173x; 1.1377x over previous
import functools

import jax
import jax.numpy as jnp
from jax import lax
from jax.experimental import pallas as pl
from jax.experimental.pallas import tpu as pltpu
import jax.experimental.pallas.tpu_sc as plsc

B, N, D, K, C = 8, 65536, 128, 64, 2
BN = 16384
NB = N // BN
NEG = -1e9


def _scorer_body(inst_ref, mask_ref, w1_ref, aux_ref, b2_ref, out_ref):
  x = inst_ref[0]
  h = jnp.maximum(jnp.dot(x, w1_ref[...], preferred_element_type=jnp.float32)
                  + aux_ref[0][None, :], 0.0)
  s2 = lax.dot_general(aux_ref[1][None, :], h, (((1,), (1,)), ((), ())),
                       preferred_element_type=jnp.float32)
  s = s2[0] + b2_ref[0]
  s = jnp.where(mask_ref[0, 0, 0] == 0, NEG, s)
  out_ref[0, 0, 0] = s


def _scores_tc(instances, mask4, W1, aux, b2):
  return pl.pallas_call(
      _scorer_body,
      grid=(B, NB),
      in_specs=[
          pl.BlockSpec((1, BN, D), lambda b, n: (b, n, 0)),
          pl.BlockSpec((1, 1, 1, BN), lambda b, n: (b, n, 0, 0)),
          pl.BlockSpec((D, D // 2), lambda b, n: (0, 0)),
          pl.BlockSpec((2, D // 2), lambda b, n: (0, 0)),
          pl.BlockSpec(memory_space=pltpu.SMEM),
      ],
      out_specs=pl.BlockSpec((1, 1, 1, BN), lambda b, n: (b, n, 0, 0)),
      out_shape=jax.ShapeDtypeStruct((B, NB, 1, BN), jnp.float32),
  )(instances, mask4, W1, aux, b2)


def _finish_body(s_ref, att_ref):
  s = s_ref[0]
  m = jnp.max(s)
  e = jnp.exp(s - m)
  att_ref[0] = e * (1.0 / jnp.sum(e))


def _finish_tc(scores3):
  return pl.pallas_call(
      _finish_body,
      grid=(B,),
      in_specs=[
          pl.BlockSpec((1, N // D, D), lambda b: (b, 0, 0)),
      ],
      out_specs=pl.BlockSpec((1, N // D, D), lambda b: (b, 0, 0)),
      out_shape=jax.ShapeDtypeStruct((B, N // D, D), jnp.float32),
  )(scores3)


GROUPS = 64
GSZ = N // GROUPS
CHUNK = GSZ // 16
SENT = -3.0e38


def _round_bf16(x):
  b = plsc.bitcast(x, jnp.int32)
  r = (b + 0x7FFF + ((b >> 16) & 1)) & jnp.int32(-65536)
  return plsc.bitcast(r, jnp.float32)


def _topk_body(scores_hbm, inst_hbm, wcb_hbm, topk_hbm, bf_hbm, log_hbm,
               sv, qsv, cm, gsm, part, shared, merged, idxout, gidx, rows,
               bfout, wcb, logout, sem, sem2):
  cid = lax.axis_index("c")
  sid = lax.axis_index("s")
  q = sid % 4
  bagl = sid // 4
  bag = cid * 4 + bagl
  iota = lax.broadcasted_iota(jnp.int32, (16,), 0)
  iota16 = iota * 16
  lane0 = iota == 0
  BIGI = jnp.int32(1 << 30)

  def full_i(x):
    return jnp.full((16,), x, jnp.int32)

  def full_f(x):
    return jnp.full((16,), x, jnp.float32)

  def _quarter_cm(src):
    pg = full_f(SENT)
    for gg in range(16):
      def _init(t, acc):
        base = gg * GSZ + t * 64
        for u in range(4):
          acc = jnp.maximum(acc, plsc.load_gather(src, [base + u * 16 + iota]))
        return acc
      acc = lax.fori_loop(0, CHUNK // 4, _init, full_f(SENT))
      part[pl.ds(gg * 16, 16)] = acc
      pg = jnp.where(iota == gg, jnp.max(acc), pg)
    part[pl.ds(256, 16)] = pg

  @pl.when(q == 0)
  def _():
    pltpu.async_copy(scores_hbm.at[bag], sv, sem2)

  pltpu.sync_copy(scores_hbm.at[bag, pl.ds(q * (N // 4), N // 4)], qsv)
  _quarter_cm(qsv)

  pltpu.sync_copy(part, shared.at[sid])
  plsc.subcore_barrier()

  @pl.when(q == 0)
  def _leader():
    pltpu.sync_copy(shared.at[pl.ds(bagl * 4, 4)], merged)
    for qq in range(4):
      for i in range(16):
        cm[pl.ds(qq * 256 + i * 16, 16)] = merged[qq, pl.ds(i * 16, 16)]
      gsm[pl.ds(qq * 16, 16)] = merged[qq, pl.ds(256, 16)]
    pltpu.make_async_copy(scores_hbm.at[bag], sv, sem2).wait()

    def _extract(t, carry):
      g0 = gsm[pl.ds(0, 16)]
      g1 = gsm[pl.ds(16, 16)]
      g2 = gsm[pl.ds(32, 16)]
      g3 = gsm[pl.ds(48, 16)]
      gm = jnp.max(jnp.maximum(jnp.maximum(g0, g1), jnp.maximum(g2, g3)))
      pm = jnp.minimum(
          jnp.minimum(jnp.where(g0 == gm, iota, BIGI),
                      jnp.where(g1 == gm, iota + 16, BIGI)),
          jnp.minimum(jnp.where(g2 == gm, iota + 32, BIGI),
                      jnp.where(g3 == gm, iota + 48, BIGI)))
      gstar = jnp.min(pm)
      cmrow = plsc.load_gather(cm, [gstar * 16 + iota])
      lstar = jnp.min(jnp.where(cmrow == gm, iota, BIGI))
      base = gstar * GSZ + lstar
      pacc = full_i(BIGI)
      vals_l, idx_l = [], []
      for u in range(4):
        idxv = base + u * 256 + iota16
        vals = plsc.load_gather(sv, [idxv])
        vals_l.append(vals)
        idx_l.append(idxv)
        pacc = jnp.minimum(pacc, jnp.where(vals == gm, idxv, BIGI))
      p = jnp.min(pacc)
      plsc.store_scatter(sv, [full_i(p)], full_f(SENT), mask=lane0)
      macc = full_f(SENT)
      for u in range(4):
        macc = jnp.maximum(macc,
                           jnp.where(idx_l[u] == p, full_f(SENT), vals_l[u]))
      newmax = jnp.max(macc)
      plsc.store_scatter(cm, [full_i(gstar * 16 + lstar)], full_f(newmax),
                         mask=lane0)
      newrow = jnp.where(iota == lstar, newmax, cmrow)
      plsc.store_scatter(gsm, [full_i(gstar)], full_f(jnp.max(newrow)),
                         mask=lane0)
      plsc.store_scatter(idxout, [full_i(t)], full_i(p), mask=lane0)
      plsc.store_scatter(gidx, [full_i(t)], full_i(p + bag * N), mask=lane0)
      return carry

    lax.fori_loop(0, K, _extract, jnp.int32(0))

    pltpu.async_copy(inst_hbm.at[gidx], rows, sem).wait()

    def _pool(r, accs):
      rv = full_i(r)
      return tuple(
          accs[c] + plsc.load_gather(rows, [rv, c * 16 + iota])
          for c in range(8))
    accs = lax.fori_loop(0, K, _pool, tuple(full_f(0.0) for _ in range(8)))
    bfv = [accs[c] * (1.0 / K) for c in range(8)]
    for c in range(8):
      bfout[pl.ds(c * 16, 16)] = bfv[c]

    pltpu.sync_copy(wcb_hbm, wcb)
    lacc0 = full_f(0.0)
    lacc1 = full_f(0.0)
    for c in range(8):
      bfr = _round_bf16(bfv[c])
      lacc0 = lacc0 + bfr * _round_bf16(wcb[pl.ds(c * 16, 16)])
      lacc1 = lacc1 + bfr * _round_bf16(wcb[pl.ds(128 + c * 16, 16)])
    l0 = jnp.sum(lacc0) + wcb[pl.ds(256, 16)][0]
    l1 = jnp.sum(lacc1) + wcb[pl.ds(256, 16)][1]
    lvec = jnp.where(iota == 0, full_f(l0),
                     jnp.where(iota == 1, full_f(l1), full_f(0.0)))
    logout[pl.ds(0, 16)] = lvec

    pltpu.sync_copy(idxout, topk_hbm.at[bag])
    pltpu.sync_copy(bfout, bf_hbm.at[bag])
    pltpu.sync_copy(logout, log_hbm.at[bag])


def _topk_sc(scores, inst_flat, wcb):
  mesh = plsc.VectorSubcoreMesh(core_axis_name="c", subcore_axis_name="s",
                                num_cores=2, num_subcores=16)
  f = pl.kernel(
      _topk_body,
      out_type=[
          jax.ShapeDtypeStruct((B, K), jnp.int32),
          jax.ShapeDtypeStruct((B, D), jnp.float32),
          jax.ShapeDtypeStruct((B, 16), jnp.float32),
      ],
      mesh=mesh,
      scratch_types=[
          pltpu.VMEM((N,), jnp.float32),
          pltpu.VMEM((N // 4,), jnp.float32),
          pltpu.VMEM((GROUPS * 16,), jnp.float32),
          pltpu.VMEM((GROUPS,), jnp.float32),
          pltpu.VMEM((272,), jnp.float32),
          pltpu.VMEM_SHARED((16, 272), jnp.float32),
          pltpu.VMEM((4, 272), jnp.float32),
          pltpu.VMEM((K,), jnp.int32),
          pltpu.VMEM((K,), jnp.int32),
          pltpu.VMEM((K, D), jnp.float32),
          pltpu.VMEM((D,), jnp.float32),
          pltpu.VMEM((2 * D + 16,), jnp.float32),
          pltpu.VMEM((16,), jnp.float32),
          pltpu.SemaphoreType.DMA,
          pltpu.SemaphoreType.DMA,
      ],
      compiler_params=pltpu.CompilerParams(use_tc_tiling_on_sc=False,
                                           needs_layout_passes=False),
  )
  return f(scores, inst_flat, wcb)


def kernel(instances, mask, W1, b1, W2, b2, Wc, bc):
  instances = jnp.asarray(instances)
  if instances.ndim == 2:
    instances = instances[None]
  mask4 = mask.reshape(B, NB, 1, BN)
  aux = jnp.stack([b1, W2[:, 0]])
  scores4 = _scores_tc(instances, mask4, W1, aux, b2)
  scores = scores4.reshape(B, N)

  wcb = jnp.concatenate([Wc.T.reshape(-1), bc,
                         jnp.zeros((16 - C,), jnp.float32)])
  topk_indices, bag_features, logpad = _topk_sc(
      scores, instances.reshape(B * N, D), wcb)

  att3 = _finish_tc(scores.reshape(B, N // D, D))
  attention_weights = att3.reshape(B, N)
  return (logpad[:, :C], attention_weights, topk_indices, bag_features)


# speedup vs baseline: 2.6031x; 1.0769x over previous
import functools

import jax
import jax.numpy as jnp
from jax import lax
from jax.experimental import pallas as pl
from jax.experimental.pallas import tpu as pltpu
import jax.experimental.pallas.tpu_sc as plsc

B, N, D, K, C = 8, 65536, 128, 64, 2
BN = 32768
NB = N // BN
NEG = -1e9


def _scorer_body(inst_ref, mask_ref, w1_ref, aux_ref, b2_ref, out_ref):
  x = inst_ref[0]
  h = jnp.maximum(jnp.dot(x, w1_ref[...], preferred_element_type=jnp.float32)
                  + aux_ref[0][None, :], 0.0)
  s2 = lax.dot_general(aux_ref[1][None, :], h, (((1,), (1,)), ((), ())),
                       preferred_element_type=jnp.float32)
  s = s2[0] + b2_ref[0]
  s = jnp.where(mask_ref[0, 0, 0] == 0, NEG, s)
  out_ref[0, 0, 0] = s


def _scores_tc(instances, mask4, W1, aux, b2):
  return pl.pallas_call(
      _scorer_body,
      grid=(B, NB),
      in_specs=[
          pl.BlockSpec((1, BN, D), lambda b, n: (b, n, 0)),
          pl.BlockSpec((1, 1, 1, BN), lambda b, n: (b, n, 0, 0)),
          pl.BlockSpec((D, D // 2), lambda b, n: (0, 0)),
          pl.BlockSpec((2, D // 2), lambda b, n: (0, 0)),
          pl.BlockSpec(memory_space=pltpu.SMEM),
      ],
      out_specs=pl.BlockSpec((1, 1, 1, BN), lambda b, n: (b, n, 0, 0)),
      out_shape=jax.ShapeDtypeStruct((B, NB, 1, BN), jnp.float32),
  )(instances, mask4, W1, aux, b2)


def _finish_body(s_ref, att_ref):
  s = s_ref[0]
  m = jnp.max(s)
  e = jnp.exp(s - m)
  att_ref[0] = e * (1.0 / jnp.sum(e))


def _finish_tc(scores3):
  return pl.pallas_call(
      _finish_body,
      grid=(B,),
      in_specs=[
          pl.BlockSpec((1, N // D, D), lambda b: (b, 0, 0)),
      ],
      out_specs=pl.BlockSpec((1, N // D, D), lambda b: (b, 0, 0)),
      out_shape=jax.ShapeDtypeStruct((B, N // D, D), jnp.float32),
  )(scores3)


GROUPS = 64
GSZ = N // GROUPS
CHUNK = GSZ // 16
SENT = -3.0e38


def _round_bf16(x):
  b = plsc.bitcast(x, jnp.int32)
  r = (b + 0x7FFF + ((b >> 16) & 1)) & jnp.int32(-65536)
  return plsc.bitcast(r, jnp.float32)


def _topk_body(scores_hbm, inst_hbm, wcb_hbm, topk_hbm, bf_hbm, log_hbm,
               sv, qsv, cm, gsm, part, shared, merged, idxout, gidx, rows,
               bfout, wcb, logout, sem, sem2):
  cid = lax.axis_index("c")
  sid = lax.axis_index("s")
  q = sid % 4
  bagl = sid // 4
  bag = cid * 4 + bagl
  iota = lax.broadcasted_iota(jnp.int32, (16,), 0)
  iota16 = iota * 16
  lane0 = iota == 0
  BIGI = jnp.int32(1 << 30)

  def full_i(x):
    return jnp.full((16,), x, jnp.int32)

  def full_f(x):
    return jnp.full((16,), x, jnp.float32)

  def _quarter_cm(src):
    pg = full_f(SENT)
    for gg in range(16):
      def _init(t, acc):
        base = gg * GSZ + t * 64
        for u in range(4):
          acc = jnp.maximum(acc, plsc.load_gather(src, [base + u * 16 + iota]))
        return acc
      acc = lax.fori_loop(0, CHUNK // 4, _init, full_f(SENT))
      part[pl.ds(gg * 16, 16)] = acc
      pg = jnp.where(iota == gg, jnp.max(acc), pg)
    part[pl.ds(256, 16)] = pg

  @pl.when(q == 0)
  def _():
    pltpu.async_copy(scores_hbm.at[bag], sv, sem2)

  pltpu.sync_copy(scores_hbm.at[bag, pl.ds(q * (N // 4), N // 4)], qsv)
  _quarter_cm(qsv)

  pltpu.sync_copy(part, shared.at[sid])
  plsc.subcore_barrier()

  @pl.when(q == 0)
  def _leader():
    pltpu.sync_copy(shared.at[pl.ds(bagl * 4, 4)], merged)
    for qq in range(4):
      for i in range(16):
        cm[pl.ds(qq * 256 + i * 16, 16)] = merged[qq, pl.ds(i * 16, 16)]
      gsm[pl.ds(qq * 16, 16)] = merged[qq, pl.ds(256, 16)]
    pltpu.make_async_copy(scores_hbm.at[bag], sv, sem2).wait()

    def _extract(t, carry):
      g0 = gsm[pl.ds(0, 16)]
      g1 = gsm[pl.ds(16, 16)]
      g2 = gsm[pl.ds(32, 16)]
      g3 = gsm[pl.ds(48, 16)]
      gm = jnp.max(jnp.maximum(jnp.maximum(g0, g1), jnp.maximum(g2, g3)))
      pm = jnp.minimum(
          jnp.minimum(jnp.where(g0 == gm, iota, BIGI),
                      jnp.where(g1 == gm, iota + 16, BIGI)),
          jnp.minimum(jnp.where(g2 == gm, iota + 32, BIGI),
                      jnp.where(g3 == gm, iota + 48, BIGI)))
      gstar = jnp.min(pm)
      cmrow = plsc.load_gather(cm, [gstar * 16 + iota])
      lstar = jnp.min(jnp.where(cmrow == gm, iota, BIGI))
      base = gstar * GSZ + lstar
      pacc = full_i(BIGI)
      vals_l, idx_l = [], []
      for u in range(4):
        idxv = base + u * 256 + iota16
        vals = plsc.load_gather(sv, [idxv])
        vals_l.append(vals)
        idx_l.append(idxv)
        pacc = jnp.minimum(pacc, jnp.where(vals == gm, idxv, BIGI))
      p = jnp.min(pacc)
      plsc.store_scatter(sv, [full_i(p)], full_f(SENT), mask=lane0)
      macc = full_f(SENT)
      for u in range(4):
        macc = jnp.maximum(macc,
                           jnp.where(idx_l[u] == p, full_f(SENT), vals_l[u]))
      newmax = jnp.max(macc)
      plsc.store_scatter(cm, [full_i(gstar * 16 + lstar)], full_f(newmax),
                         mask=lane0)
      newrow = jnp.where(iota == lstar, newmax, cmrow)
      plsc.store_scatter(gsm, [full_i(gstar)], full_f(jnp.max(newrow)),
                         mask=lane0)
      plsc.store_scatter(idxout, [full_i(t)], full_i(p), mask=lane0)
      plsc.store_scatter(gidx, [full_i(t)], full_i(p + bag * N), mask=lane0)
      return carry

    lax.fori_loop(0, K, _extract, jnp.int32(0))

    pltpu.async_copy(inst_hbm.at[gidx], rows, sem).wait()

    def _pool(r, accs):
      rv = full_i(r)
      return tuple(
          accs[c] + plsc.load_gather(rows, [rv, c * 16 + iota])
          for c in range(8))
    accs = lax.fori_loop(0, K, _pool, tuple(full_f(0.0) for _ in range(8)))
    bfv = [accs[c] * (1.0 / K) for c in range(8)]
    for c in range(8):
      bfout[pl.ds(c * 16, 16)] = bfv[c]

    pltpu.sync_copy(wcb_hbm, wcb)
    lacc0 = full_f(0.0)
    lacc1 = full_f(0.0)
    for c in range(8):
      bfr = _round_bf16(bfv[c])
      lacc0 = lacc0 + bfr * _round_bf16(wcb[pl.ds(c * 16, 16)])
      lacc1 = lacc1 + bfr * _round_bf16(wcb[pl.ds(128 + c * 16, 16)])
    l0 = jnp.sum(lacc0) + wcb[pl.ds(256, 16)][0]
    l1 = jnp.sum(lacc1) + wcb[pl.ds(256, 16)][1]
    lvec = jnp.where(iota == 0, full_f(l0),
                     jnp.where(iota == 1, full_f(l1), full_f(0.0)))
    logout[pl.ds(0, 16)] = lvec

    pltpu.sync_copy(idxout, topk_hbm.at[bag])
    pltpu.sync_copy(bfout, bf_hbm.at[bag])
    pltpu.sync_copy(logout, log_hbm.at[bag])


def _topk_sc(scores, inst_flat, wcb):
  mesh = plsc.VectorSubcoreMesh(core_axis_name="c", subcore_axis_name="s",
                                num_cores=2, num_subcores=16)
  f = pl.kernel(
      _topk_body,
      out_type=[
          jax.ShapeDtypeStruct((B, K), jnp.int32),
          jax.ShapeDtypeStruct((B, D), jnp.float32),
          jax.ShapeDtypeStruct((B, 16), jnp.float32),
      ],
      mesh=mesh,
      scratch_types=[
          pltpu.VMEM((N,), jnp.float32),
          pltpu.VMEM((N // 4,), jnp.float32),
          pltpu.VMEM((GROUPS * 16,), jnp.float32),
          pltpu.VMEM((GROUPS,), jnp.float32),
          pltpu.VMEM((272,), jnp.float32),
          pltpu.VMEM_SHARED((16, 272), jnp.float32),
          pltpu.VMEM((4, 272), jnp.float32),
          pltpu.VMEM((K,), jnp.int32),
          pltpu.VMEM((K,), jnp.int32),
          pltpu.VMEM((K, D), jnp.float32),
          pltpu.VMEM((D,), jnp.float32),
          pltpu.VMEM((2 * D + 16,), jnp.float32),
          pltpu.VMEM((16,), jnp.float32),
          pltpu.SemaphoreType.DMA,
          pltpu.SemaphoreType.DMA,
      ],
      compiler_params=pltpu.CompilerParams(use_tc_tiling_on_sc=False,
                                           needs_layout_passes=False),
  )
  return f(scores, inst_flat, wcb)


def kernel(instances, mask, W1, b1, W2, b2, Wc, bc):
  instances = jnp.asarray(instances)
  if instances.ndim == 2:
    instances = instances[None]
  mask4 = mask.reshape(B, NB, 1, BN)
  aux = jnp.stack([b1, W2[:, 0]])
  scores4 = _scores_tc(instances, mask4, W1, aux, b2)
  scores = scores4.reshape(B, N)

  wcb = jnp.concatenate([Wc.T.reshape(-1), bc,
                         jnp.zeros((16 - C,), jnp.float32)])
  topk_indices, bag_features, logpad = _topk_sc(
      scores, instances.reshape(B * N, D), wcb)

  att3 = _finish_tc(scores.reshape(B, N // D, D))
  attention_weights = att3.reshape(B, N)
  return (logpad[:, :C], attention_weights, topk_indices, bag_features)
